# Initial kernel scaffold; baseline (speedup 1.0000x reference)
import jax
import jax.numpy as jnp
from jax.experimental import pallas as pl


def kernel(idx, token_emb, pos_emb, ln1_w, ln1_b, ln2_w, ln2_b, W_attn, W_proj, W_router, W_sh_up, W_sh_down, W_up, W_down):
    raise NotImplementedError("write your pallas kernel here")


# speedup vs baseline: 2.7556x; 2.7556x over previous
import jax
import jax.numpy as jnp
from jax import lax
from jax.experimental import pallas as pl
from jax.experimental.pallas import tpu as pltpu
from jax.experimental.pallas import tpu_sc as plsc

_B, _T, _D, _H, _DH = 1, 2048, 768, 12, 64
_V, _E, _EDIM, _K = 50304, 16, 512, 2
_TB = 256
_NTB = _T // _TB

_NC, _NS = 2, 16
_NW = _NC * _NS
_RPW = _T // _NW


def _emb_gather_body(table_hbm, idx_hbm, out_hbm, idx_v, rows_v, sem):
    wid = lax.axis_index("s") * _NC + lax.axis_index("c")
    base = wid * _RPW
    pltpu.sync_copy(idx_hbm.at[pl.ds(base, _RPW)], idx_v)
    pltpu.async_copy(table_hbm.at[idx_v], rows_v, sem).wait()
    pltpu.sync_copy(rows_v, out_hbm.at[pl.ds(base, _RPW)])


def _emb_gather(table, idx_flat):
    mesh = plsc.VectorSubcoreMesh(core_axis_name="c", subcore_axis_name="s")
    k = pl.kernel(
        _emb_gather_body,
        mesh=mesh,
        out_type=jax.ShapeDtypeStruct((_T, _D), jnp.float32),
        scratch_types=[
            pltpu.VMEM((_RPW,), jnp.int32),
            pltpu.VMEM((_RPW, _D), jnp.float32),
            pltpu.SemaphoreType.DMA,
        ],
    )
    return k(table, idx_flat)


def _ln(h, w, b):
    mu = jnp.mean(h, axis=-1, keepdims=True)
    var = jnp.mean((h - mu) ** 2, axis=-1, keepdims=True)
    return (h - mu) * lax.rsqrt(var + 1e-5) * w + b


def _gelu(x):
    return 0.5 * x * (1.0 + lax.erf(x * 0.7071067811865476))


def _dot_t(a, b):
    return lax.dot_general(a, b, (((1,), (1,)), ((), ())),
                           preferred_element_type=jnp.float32)


def _preattn_body(emb_ref, pos_ref, w1_ref, b1_ref, wattn_ref, x_ref, qkv_ref):
    x = emb_ref[...] + pos_ref[...]
    x_ref[...] = x
    h = _ln(x, w1_ref[...], b1_ref[...])
    qkv_ref[...] = _dot_t(h, wattn_ref[...])


def _attn_body(qkv_ref, y_ref):
    i = pl.program_id(0)
    scale = 1.0 / jnp.sqrt(jnp.float32(_DH))
    rows = i * _TB + lax.broadcasted_iota(jnp.int32, (_TB, _T), 0)
    cols = lax.broadcasted_iota(jnp.int32, (_TB, _T), 1)
    mask = cols <= rows
    neg = jnp.finfo(jnp.float32).min
    for h in range(_H):
        q = qkv_ref[pl.ds(i * _TB, _TB), h * _DH:(h + 1) * _DH]
        kk = qkv_ref[:, _D + h * _DH:_D + (h + 1) * _DH]
        v = qkv_ref[:, 2 * _D + h * _DH:2 * _D + (h + 1) * _DH]
        s = _dot_t(q, kk) * scale
        s = jnp.where(mask, s, neg)
        m = jnp.max(s, axis=-1, keepdims=True)
        p = jnp.exp(s - m)
        p = p / jnp.sum(p, axis=-1, keepdims=True)
        y_ref[:, h * _DH:(h + 1) * _DH] = lax.dot_general(
            p, v, (((1,), (0,)), ((), ())), preferred_element_type=jnp.float32)


def _postattn_body(y_ref, x_ref, w2_ref, b2_ref, wproj_ref, wshup_ref,
                   wshdn_ref, wrout_ref, h2_ref, base_ref, gate_ref):
    x = x_ref[...]
    attn = _dot_t(y_ref[...], wproj_ref[...]) + x
    h2 = _ln(attn, w2_ref[...], b2_ref[...])
    h2_ref[...] = h2
    up = _dot_t(h2, wshup_ref[...])
    shared = _dot_t(_gelu(up), wshdn_ref[...])
    base_ref[...] = shared + x
    logits = _dot_t(h2, wrout_ref[...])
    lm = jnp.max(logits, axis=-1, keepdims=True)
    pe = jnp.exp(logits - lm)
    probs = pe / jnp.sum(pe, axis=-1, keepdims=True)
    col = lax.broadcasted_iota(jnp.int32, (_TB, _E), 1)
    m1 = jnp.max(probs, axis=-1, keepdims=True)
    i1 = jnp.min(jnp.where(probs == m1, col, _E), axis=-1, keepdims=True)
    p2 = jnp.where(col == i1, -1.0, probs)
    m2 = jnp.max(p2, axis=-1, keepdims=True)
    i2 = jnp.min(jnp.where((probs == m2) & (col != i1), col, _E),
                 axis=-1, keepdims=True)
    gate_ref[...] = jnp.where((col == i1) | (col == i2), probs, 0.0)


def _moe_body(h2_ref, base_ref, gate_ref, wup_ref, wdn_ref, out_ref):
    e = pl.program_id(0)

    @pl.when(e == 0)
    def _():
        out_ref[...] = base_ref[...]

    emask = (lax.broadcasted_iota(jnp.int32, (1, _E), 1) == e).astype(jnp.float32)
    gcol = jnp.sum(gate_ref[...] * emask, axis=1, keepdims=True)
    up = _dot_t(h2_ref[...], wup_ref[0])
    dn = _dot_t(_gelu(up), wdn_ref[0])
    out_ref[...] += dn * gcol


def kernel(idx, token_emb, pos_emb, ln1_w, ln1_b, ln2_w, ln2_b, W_attn,
           W_proj, W_router, W_sh_up, W_sh_down, W_up, W_down):
    idx_flat = idx.reshape(_T).astype(jnp.int32)
    emb = _emb_gather(token_emb, idx_flat)

    x, qkv = pl.pallas_call(
        _preattn_body,
        grid=(_NTB,),
        in_specs=[
            pl.BlockSpec((_TB, _D), lambda i: (i, 0)),
            pl.BlockSpec((_TB, _D), lambda i: (i, 0)),
            pl.BlockSpec((1, _D), lambda i: (0, 0)),
            pl.BlockSpec((1, _D), lambda i: (0, 0)),
            pl.BlockSpec((3 * _D, _D), lambda i: (0, 0)),
        ],
        out_specs=[
            pl.BlockSpec((_TB, _D), lambda i: (i, 0)),
            pl.BlockSpec((_TB, 3 * _D), lambda i: (i, 0)),
        ],
        out_shape=[
            jax.ShapeDtypeStruct((_T, _D), jnp.float32),
            jax.ShapeDtypeStruct((_T, 3 * _D), jnp.float32),
        ],
    )(emb, pos_emb, ln1_w.reshape(1, _D), ln1_b.reshape(1, _D), W_attn)

    y = pl.pallas_call(
        _attn_body,
        grid=(_NTB,),
        in_specs=[pl.BlockSpec((_T, 3 * _D), lambda i: (0, 0))],
        out_specs=pl.BlockSpec((_TB, _D), lambda i: (i, 0)),
        out_shape=jax.ShapeDtypeStruct((_T, _D), jnp.float32),
    )(qkv)

    h2, base, gate = pl.pallas_call(
        _postattn_body,
        grid=(_NTB,),
        in_specs=[
            pl.BlockSpec((_TB, _D), lambda i: (i, 0)),
            pl.BlockSpec((_TB, _D), lambda i: (i, 0)),
            pl.BlockSpec((1, _D), lambda i: (0, 0)),
            pl.BlockSpec((1, _D), lambda i: (0, 0)),
            pl.BlockSpec((_D, _D), lambda i: (0, 0)),
            pl.BlockSpec((_EDIM, _D), lambda i: (0, 0)),
            pl.BlockSpec((_D, _EDIM), lambda i: (0, 0)),
            pl.BlockSpec((_E, _D), lambda i: (0, 0)),
        ],
        out_specs=[
            pl.BlockSpec((_TB, _D), lambda i: (i, 0)),
            pl.BlockSpec((_TB, _D), lambda i: (i, 0)),
            pl.BlockSpec((_TB, _E), lambda i: (i, 0)),
        ],
        out_shape=[
            jax.ShapeDtypeStruct((_T, _D), jnp.float32),
            jax.ShapeDtypeStruct((_T, _D), jnp.float32),
            jax.ShapeDtypeStruct((_T, _E), jnp.float32),
        ],
    )(y, x, ln2_w.reshape(1, _D), ln2_b.reshape(1, _D), W_proj, W_sh_up,
      W_sh_down, W_router)

    out = pl.pallas_call(
        _moe_body,
        grid=(_E,),
        in_specs=[
            pl.BlockSpec((_T, _D), lambda e: (0, 0)),
            pl.BlockSpec((_T, _D), lambda e: (0, 0)),
            pl.BlockSpec((_T, _E), lambda e: (0, 0)),
            pl.BlockSpec((1, _EDIM, _D), lambda e: (e, 0, 0)),
            pl.BlockSpec((1, _D, _EDIM), lambda e: (e, 0, 0)),
        ],
        out_specs=pl.BlockSpec((_T, _D), lambda e: (0, 0)),
        out_shape=jax.ShapeDtypeStruct((_T, _D), jnp.float32),
        compiler_params=pltpu.CompilerParams(
            dimension_semantics=("arbitrary",)),
    )(h2, base, gate, W_up, W_down)

    return out.reshape(_B, _T, _D)
